# Initial kernel scaffold; baseline (speedup 1.0000x reference)
import jax
import jax.numpy as jnp
from jax.experimental import pallas as pl


def kernel(y_points, y_feats, x_points, x_feats, W0, b0, gamma0, beta0, W1, b1, gamma1, beta1, W2, b2, gamma2, beta2):
    raise NotImplementedError("write your pallas kernel here")


# speedup vs baseline: 11.0344x; 11.0344x over previous
import functools

import jax
import jax.numpy as jnp
from jax import lax
from jax.experimental import pallas as pl
from jax.experimental.pallas import tpu as pltpu
from jax.experimental.pallas import tpu_sc as plsc

B, N, M = 8, 4096, 1024
CY, CX = 256, 128
P = B * N
PB = 512
NB = N // PB
BIG = 3.0e38

NC, NS = 2, 16
NW = NC * NS
QPW = P // NW
T = 64
STEPS = QPW // T


def _knn_body(xp_ref, yp_ref, idx_ref, w_ref):
    b = pl.program_id(0)
    xp = xp_ref[0]
    yp = yp_ref[0]
    x2 = jnp.sum(xp * xp, axis=1, keepdims=True)
    one_x = jnp.ones((PB, 1), jnp.float32)
    x_aug = jnp.concatenate([xp, x2, one_x], axis=1)
    y2 = jnp.sum(yp * yp, axis=1, keepdims=True)
    one_y = jnp.ones((M, 1), jnp.float32)
    y_aug = jnp.concatenate([-2.0 * yp, one_y, y2], axis=1)
    d2t = lax.dot_general(y_aug, x_aug, (((1,), (1,)), ((), ())),
                          precision=lax.Precision.HIGHEST,
                          preferred_element_type=jnp.float32)
    d2t = jnp.maximum(d2t, 0.0)
    iota = lax.broadcasted_iota(jnp.int32, (M, PB), 0)
    vals, idxs = [], []
    for k in range(3):
        mk = jnp.min(d2t, axis=0, keepdims=True)
        ik = jnp.min(jnp.where(d2t == mk, iota, M), axis=0, keepdims=True)
        vals.append(mk)
        idxs.append(ik)
        if k < 2:
            d2t = jnp.where(iota == ik, BIG, d2t)
    r = [1.0 / (v + 1e-8) for v in vals]
    rs = r[0] + r[1] + r[2]
    base = b * M
    one16 = jnp.ones((1, 16), jnp.float32)
    for k in range(3):
        idx_ref[k:k + 1, :] = idxs[k] + base
        w_ref[k] = lax.dot_general(r[k] / rs, one16, (((0,), (0,)), ((), ())),
                                   preferred_element_type=jnp.float32)


def _knn(x_points, y_points):
    return pl.pallas_call(
        _knn_body,
        grid=(B, NB),
        in_specs=[
            pl.BlockSpec((1, PB, 3), lambda b, nb: (b, nb, 0)),
            pl.BlockSpec((1, M, 3), lambda b, nb: (b, 0, 0)),
        ],
        out_specs=[
            pl.BlockSpec((3, PB), lambda b, nb: (0, b * NB + nb)),
            pl.BlockSpec((3, PB, 16), lambda b, nb: (0, b * NB + nb, 0)),
        ],
        out_shape=[
            jax.ShapeDtypeStruct((3, P), jnp.int32),
            jax.ShapeDtypeStruct((3, P, 16), jnp.float32),
        ],
    )(x_points, y_points)


def _sc_interp_body(yf_ref, idx_ref, w_ref, out_ref,
                    i0_v, i1_v, i2_v, w0_v, w1_v, w2_v,
                    r0_v, r1_v, r2_v, acc_v, sem):
    wid = lax.axis_index("s") * NC + lax.axis_index("c")
    base0 = wid * QPW

    def step(t, carry):
        base = base0 + t * T
        pltpu.sync_copy(w_ref.at[0, pl.ds(base, T)], w0_v)
        pltpu.sync_copy(w_ref.at[1, pl.ds(base, T)], w1_v)
        pltpu.sync_copy(w_ref.at[2, pl.ds(base, T)], w2_v)
        pltpu.sync_copy(idx_ref.at[0, pl.ds(base, T)], i0_v)
        pltpu.sync_copy(idx_ref.at[1, pl.ds(base, T)], i1_v)
        pltpu.sync_copy(idx_ref.at[2, pl.ds(base, T)], i2_v)
        c0 = pltpu.async_copy(yf_ref.at[i0_v], r0_v, sem)
        c1 = pltpu.async_copy(yf_ref.at[i1_v], r1_v, sem)
        c2 = pltpu.async_copy(yf_ref.at[i2_v], r2_v, sem)
        c0.wait()
        c1.wait()
        c2.wait()

        def q_body(q, carry2):
            wv0 = w0_v[q, :]
            wv1 = w1_v[q, :]
            wv2 = w2_v[q, :]
            for c in range(CY // 16):
                sl = pl.ds(c * 16, 16)
                acc_v[q, sl] = (r0_v[q, sl] * wv0 + r1_v[q, sl] * wv1
                                + r2_v[q, sl] * wv2)
            return carry2

        lax.fori_loop(0, T, q_body, 0)
        pltpu.sync_copy(acc_v, out_ref.at[pl.ds(base, T)])
        return carry

    lax.fori_loop(0, STEPS, step, 0)


def _sc_interp(yf_flat, idx, w):
    kfn = functools.partial(
        pl.kernel,
        out_type=jax.ShapeDtypeStruct((P, CY), jnp.float32),
        mesh=plsc.VectorSubcoreMesh(core_axis_name="c", subcore_axis_name="s"),
        scratch_types=[
            pltpu.VMEM((T,), jnp.int32),
            pltpu.VMEM((T,), jnp.int32),
            pltpu.VMEM((T,), jnp.int32),
            pltpu.VMEM((T, 16), jnp.float32),
            pltpu.VMEM((T, 16), jnp.float32),
            pltpu.VMEM((T, 16), jnp.float32),
            pltpu.VMEM((T, CY), jnp.float32),
            pltpu.VMEM((T, CY), jnp.float32),
            pltpu.VMEM((T, CY), jnp.float32),
            pltpu.VMEM((T, CY), jnp.float32),
            pltpu.SemaphoreType.DMA,
        ],
    )(_sc_interp_body)
    return kfn(yf_flat, idx, w)


def _l0_body(a_ref, xf_ref, wa_ref, wb_ref, y_ref, st_ref, acc):
    step = pl.program_id(0)

    @pl.when(step == 0)
    def _():
        acc[...] = jnp.zeros_like(acc)

    y = lax.dot_general(a_ref[...], wa_ref[...], (((1,), (1,)), ((), ())),
                        preferred_element_type=jnp.float32)
    y += lax.dot_general(xf_ref[...], wb_ref[...], (((1,), (1,)), ((), ())),
                         preferred_element_type=jnp.float32)
    y_ref[...] = y
    acc[0:1, :] += jnp.sum(y, axis=0, keepdims=True)
    acc[1:2, :] += jnp.sum(y * y, axis=0, keepdims=True)

    @pl.when(step == pl.num_programs(0) - 1)
    def _():
        st_ref[...] = acc[...]


def _layer0(interp, xf, W0a, W0b):
    cout = W0a.shape[0]
    return pl.pallas_call(
        _l0_body,
        grid=(P // PB,),
        in_specs=[
            pl.BlockSpec((PB, CY), lambda s: (s, 0)),
            pl.BlockSpec((PB, CX), lambda s: (s, 0)),
            pl.BlockSpec((cout, CY), lambda s: (0, 0)),
            pl.BlockSpec((cout, CX), lambda s: (0, 0)),
        ],
        out_specs=[
            pl.BlockSpec((PB, cout), lambda s: (s, 0)),
            pl.BlockSpec((8, cout), lambda s: (0, 0)),
        ],
        out_shape=[
            jax.ShapeDtypeStruct((P, cout), jnp.float32),
            jax.ShapeDtypeStruct((8, cout), jnp.float32),
        ],
        scratch_shapes=[pltpu.VMEM((8, cout), jnp.float32)],
    )(interp, xf, W0a, W0b)


def _layer_body(y_ref, ss_ref, w_ref, out_ref, st_ref, acc):
    step = pl.program_id(0)

    @pl.when(step == 0)
    def _():
        acc[...] = jnp.zeros_like(acc)

    a = jnp.maximum(y_ref[...] * ss_ref[0:1, :] + ss_ref[1:2, :], 0.0)
    y = lax.dot_general(a, w_ref[...], (((1,), (1,)), ((), ())),
                        preferred_element_type=jnp.float32)
    out_ref[...] = y
    acc[0:1, :] += jnp.sum(y, axis=0, keepdims=True)
    acc[1:2, :] += jnp.sum(y * y, axis=0, keepdims=True)

    @pl.when(step == pl.num_programs(0) - 1)
    def _():
        st_ref[...] = acc[...]


def _layer(y, ss, W):
    cout, cin = W.shape
    return pl.pallas_call(
        _layer_body,
        grid=(P // PB,),
        in_specs=[
            pl.BlockSpec((PB, cin), lambda s: (s, 0)),
            pl.BlockSpec((8, cin), lambda s: (0, 0)),
            pl.BlockSpec((cout, cin), lambda s: (0, 0)),
        ],
        out_specs=[
            pl.BlockSpec((PB, cout), lambda s: (s, 0)),
            pl.BlockSpec((8, cout), lambda s: (0, 0)),
        ],
        out_shape=[
            jax.ShapeDtypeStruct((P, cout), jnp.float32),
            jax.ShapeDtypeStruct((8, cout), jnp.float32),
        ],
        scratch_shapes=[pltpu.VMEM((8, cout), jnp.float32)],
    )(y, ss, W)


def _final_body(y_ref, ss_ref, eye_ref, out_ref):
    a = jnp.maximum(y_ref[...] * ss_ref[0:1, :] + ss_ref[1:2, :], 0.0)
    out_ref[0] = lax.dot_general(eye_ref[...], a, (((1,), (1,)), ((), ())),
                                 preferred_element_type=jnp.float32)


def _final(y2, ss, cout):
    eye = jnp.eye(cout, dtype=jnp.float32)
    return pl.pallas_call(
        _final_body,
        grid=(B, NB),
        in_specs=[
            pl.BlockSpec((PB, cout), lambda b, nb: (b * NB + nb, 0)),
            pl.BlockSpec((8, cout), lambda b, nb: (0, 0)),
            pl.BlockSpec((cout, cout), lambda b, nb: (0, 0)),
        ],
        out_specs=pl.BlockSpec((1, cout, PB), lambda b, nb: (b, 0, nb)),
        out_shape=jax.ShapeDtypeStruct((B, cout, N), jnp.float32),
    )(y2, ss, eye)


def _bn_scale_shift(st, gamma, beta):
    mean = st[0] / P
    var = st[1] / P - mean * mean
    s = gamma / jnp.sqrt(var + 1e-5)
    t = beta - mean * s
    return jnp.concatenate(
        [s[None, :], t[None, :], jnp.zeros((6, s.shape[0]), jnp.float32)], axis=0)


def kernel(y_points, y_feats, x_points, x_feats,
           W0, b0, gamma0, beta0,
           W1, b1, gamma1, beta1,
           W2, b2, gamma2, beta2):
    idx, w = _knn(x_points, y_points)
    interp = _sc_interp(y_feats.reshape(B * M, CY), idx, w)
    y0, st0 = _layer0(interp, x_feats.reshape(P, CX), W0[:, :CY], W0[:, CY:])
    ss0 = _bn_scale_shift(st0, gamma0, beta0)
    y1, st1 = _layer(y0, ss0, W1)
    ss1 = _bn_scale_shift(st1, gamma1, beta1)
    y2, st2 = _layer(y1, ss1, W2)
    ss2 = _bn_scale_shift(st2, gamma2, beta2)
    return _final(y2, ss2, W2.shape[0])


# speedup vs baseline: 13.7054x; 1.2421x over previous
import functools

import jax
import jax.numpy as jnp
from jax import lax
from jax.experimental import pallas as pl
from jax.experimental.pallas import tpu as pltpu
from jax.experimental.pallas import tpu_sc as plsc

B, N, M = 8, 4096, 1024
CY, CX = 256, 128
P = B * N
PB = 512
NB = N // PB
BIG = 3.0e38

NC, NS = 2, 16
NW = NC * NS
QPW = P // NW
T = 32
STEPS = QPW // T


def _knn_body(xp_ref, yp_ref, idx_ref, w_ref):
    b = pl.program_id(0)
    xp = xp_ref[0]
    yp = yp_ref[0]
    x2 = jnp.sum(xp * xp, axis=1, keepdims=True)
    one_x = jnp.ones((PB, 1), jnp.float32)
    x_aug = jnp.concatenate([xp, x2, one_x], axis=1)
    y2 = jnp.sum(yp * yp, axis=1, keepdims=True)
    one_y = jnp.ones((M, 1), jnp.float32)
    y_aug = jnp.concatenate([-2.0 * yp, one_y, y2], axis=1)
    d2t = lax.dot_general(y_aug, x_aug, (((1,), (1,)), ((), ())),
                          precision=lax.Precision.HIGHEST,
                          preferred_element_type=jnp.float32)
    d2t = jnp.maximum(d2t, 0.0)
    vals, idxs = [], []
    for k in range(3):
        mk = jnp.min(d2t, axis=0, keepdims=True)
        ik = jnp.argmin(d2t, axis=0).astype(jnp.int32)[None, :]
        vals.append(mk)
        idxs.append(ik)
        if k < 2:
            d2t = jnp.where(d2t <= mk, BIG, d2t)
    r = [1.0 / (v + 1e-8) for v in vals]
    rs = r[0] + r[1] + r[2]
    base = b * M
    one16 = jnp.ones((1, 16), jnp.float32)
    for k in range(3):
        idx_ref[k:k + 1, :] = idxs[k] + base
        w_ref[k] = lax.dot_general(r[k] / rs, one16, (((0,), (0,)), ((), ())),
                                   preferred_element_type=jnp.float32)


NBH = NB // 2
PH = P // 2


def _knn(x_points, y_points, h):
    return pl.pallas_call(
        _knn_body,
        grid=(B, NBH),
        in_specs=[
            pl.BlockSpec((1, PB, 3), lambda b, nb: (b, h * NBH + nb, 0)),
            pl.BlockSpec((1, M, 3), lambda b, nb: (b, 0, 0)),
        ],
        out_specs=[
            pl.BlockSpec((3, PB), lambda b, nb: (0, b * NBH + nb)),
            pl.BlockSpec((3, PB, 16), lambda b, nb: (0, b * NBH + nb, 0)),
        ],
        out_shape=[
            jax.ShapeDtypeStruct((3, PH), jnp.int32),
            jax.ShapeDtypeStruct((3, PH, 16), jnp.float32),
        ],
    )(x_points, y_points)


QPWH = PH // NW
STEPSH = QPWH // T


def _sc_interp_body(yf_ref, idx_ref, w_ref, out_ref, *scr):
    bufs = (scr[0:10], scr[10:20])
    sems = scr[20:22]
    wid = lax.axis_index("s") * NC + lax.axis_index("c")
    base0 = wid * QPWH
    obase0 = base0

    def issue(b, s):
        i0, i1, i2, w0, w1, w2, r0, r1, r2, acc = bufs[b]
        base = base0 + s * T
        pltpu.sync_copy(w_ref.at[0, pl.ds(base, T)], w0)
        pltpu.sync_copy(w_ref.at[1, pl.ds(base, T)], w1)
        pltpu.sync_copy(w_ref.at[2, pl.ds(base, T)], w2)
        pltpu.sync_copy(idx_ref.at[0, pl.ds(base, T)], i0)
        pltpu.sync_copy(idx_ref.at[1, pl.ds(base, T)], i1)
        pltpu.sync_copy(idx_ref.at[2, pl.ds(base, T)], i2)
        pltpu.async_copy(yf_ref.at[i0], r0, sems[b])
        pltpu.async_copy(yf_ref.at[i1], r1, sems[b])
        pltpu.async_copy(yf_ref.at[i2], r2, sems[b])

    def drain(b):
        i0, i1, i2, w0, w1, w2, r0, r1, r2, acc = bufs[b]
        pltpu.make_async_copy(yf_ref.at[i0], r0, sems[b]).wait()
        pltpu.make_async_copy(yf_ref.at[i1], r1, sems[b]).wait()
        pltpu.make_async_copy(yf_ref.at[i2], r2, sems[b]).wait()

    def compute(b, s):
        i0, i1, i2, w0, w1, w2, r0, r1, r2, acc = bufs[b]
        obase = obase0 + s * T

        def q_body(q, carry2):
            wv0 = w0[q, :]
            wv1 = w1[q, :]
            wv2 = w2[q, :]
            for c in range(CY // 16):
                sl = pl.ds(c * 16, 16)
                acc[q, sl] = (r0[q, sl] * wv0 + r1[q, sl] * wv1
                              + r2[q, sl] * wv2)
            return carry2

        lax.fori_loop(0, T, q_body, 0)
        pltpu.sync_copy(acc, out_ref.at[pl.ds(obase, T)])

    issue(0, 0)
    issue(1, 1)

    def step2(t2, carry):
        s = t2 * 2
        for b in range(2):
            drain(b)
            compute(b, s + b)

            @pl.when(s + b + 2 < STEPSH)
            def _():
                issue(b, s + b + 2)
        return carry

    lax.fori_loop(0, STEPSH // 2, step2, 0)


def _sc_interp(yf_flat, idx, w):
    bufset = [
        pltpu.VMEM((T,), jnp.int32),
        pltpu.VMEM((T,), jnp.int32),
        pltpu.VMEM((T,), jnp.int32),
        pltpu.VMEM((T, 16), jnp.float32),
        pltpu.VMEM((T, 16), jnp.float32),
        pltpu.VMEM((T, 16), jnp.float32),
        pltpu.VMEM((T, CY), jnp.float32),
        pltpu.VMEM((T, CY), jnp.float32),
        pltpu.VMEM((T, CY), jnp.float32),
        pltpu.VMEM((T, CY), jnp.float32),
    ]
    kfn = functools.partial(
        pl.kernel,
        out_type=jax.ShapeDtypeStruct((PH, CY), jnp.float32),
        mesh=plsc.VectorSubcoreMesh(core_axis_name="c", subcore_axis_name="s"),
        scratch_types=bufset + bufset
        + [pltpu.SemaphoreType.DMA, pltpu.SemaphoreType.DMA],
    )(_sc_interp_body)
    return kfn(yf_flat, idx, w)


def _l0_body(a0_ref, a1_ref, xf_ref, wa_ref, wb_ref, y_ref, st_ref, acc):
    step = pl.program_id(0)

    @pl.when(step == 0)
    def _():
        acc[...] = jnp.zeros_like(acc)

    def do(a_ref):
        a = a_ref[...].astype(jnp.bfloat16)
        xf = xf_ref[...].astype(jnp.bfloat16)
        y = lax.dot_general(a, wa_ref[...], (((1,), (1,)), ((), ())),
                            preferred_element_type=jnp.float32)
        y += lax.dot_general(xf, wb_ref[...], (((1,), (1,)), ((), ())),
                             preferred_element_type=jnp.float32)
        y_ref[...] = y.astype(jnp.bfloat16)
        acc[0:1, :] += jnp.sum(y, axis=0, keepdims=True)
        acc[1:2, :] += jnp.sum(y * y, axis=0, keepdims=True)

    @pl.when(step < P // PB // 2)
    def _():
        do(a0_ref)

    @pl.when(step >= P // PB // 2)
    def _():
        do(a1_ref)

    @pl.when(step == pl.num_programs(0) - 1)
    def _():
        st_ref[...] = acc[...]


def _layer0(interp0, interp1, xf, W0a, W0b):
    cout = W0a.shape[0]
    nh = P // PB // 2
    return pl.pallas_call(
        _l0_body,
        grid=(P // PB,),
        in_specs=[
            pl.BlockSpec((PB, CY), lambda s: (jnp.minimum(s, nh - 1), 0)),
            pl.BlockSpec((PB, CY), lambda s: (jnp.maximum(s - nh, 0), 0)),
            pl.BlockSpec((PB, CX),
                         lambda s: (((s % nh) // NBH) * NB
                                    + (s // nh) * NBH + s % NBH, 0)),
            pl.BlockSpec((cout, CY), lambda s: (0, 0)),
            pl.BlockSpec((cout, CX), lambda s: (0, 0)),
        ],
        out_specs=[
            pl.BlockSpec((PB, cout), lambda s: (s, 0)),
            pl.BlockSpec((8, cout), lambda s: (0, 0)),
        ],
        out_shape=[
            jax.ShapeDtypeStruct((P, cout), jnp.bfloat16),
            jax.ShapeDtypeStruct((8, cout), jnp.float32),
        ],
        scratch_shapes=[pltpu.VMEM((8, cout), jnp.float32)],
    )(interp0, interp1, xf, W0a, W0b)


def _layer_body(y_ref, ss_ref, w_ref, out_ref, st_ref, acc):
    step = pl.program_id(0)

    @pl.when(step == 0)
    def _():
        acc[...] = jnp.zeros_like(acc)

    yin = y_ref[...].astype(jnp.float32)
    a = jnp.maximum(yin * ss_ref[0:1, :] + ss_ref[1:2, :], 0.0)
    y = lax.dot_general(a.astype(jnp.bfloat16), w_ref[...],
                        (((1,), (1,)), ((), ())),
                        preferred_element_type=jnp.float32)
    out_ref[...] = y.astype(jnp.bfloat16)
    acc[0:1, :] += jnp.sum(y, axis=0, keepdims=True)
    acc[1:2, :] += jnp.sum(y * y, axis=0, keepdims=True)

    @pl.when(step == pl.num_programs(0) - 1)
    def _():
        st_ref[...] = acc[...]


def _layer(y, ss, W):
    cout, cin = W.shape
    return pl.pallas_call(
        _layer_body,
        grid=(P // PB,),
        in_specs=[
            pl.BlockSpec((PB, cin), lambda s: (s, 0)),
            pl.BlockSpec((8, cin), lambda s: (0, 0)),
            pl.BlockSpec((cout, cin), lambda s: (0, 0)),
        ],
        out_specs=[
            pl.BlockSpec((PB, cout), lambda s: (s, 0)),
            pl.BlockSpec((8, cout), lambda s: (0, 0)),
        ],
        out_shape=[
            jax.ShapeDtypeStruct((P, cout), jnp.bfloat16),
            jax.ShapeDtypeStruct((8, cout), jnp.float32),
        ],
        scratch_shapes=[pltpu.VMEM((8, cout), jnp.float32)],
    )(y, ss, W)


def _final_body(y_ref, ss_ref, eye_ref, out_ref):
    yin = y_ref[...].astype(jnp.float32)
    a = jnp.maximum(yin * ss_ref[0:1, :] + ss_ref[1:2, :], 0.0)
    out_ref[0] = lax.dot_general(eye_ref[...], a, (((1,), (1,)), ((), ())),
                                 preferred_element_type=jnp.float32)


def _final(y2, ss, cout):
    eye = jnp.eye(cout, dtype=jnp.float32)
    return pl.pallas_call(
        _final_body,
        grid=(B, NB),
        in_specs=[
            pl.BlockSpec((PB, cout),
                         lambda b, nb: ((nb // NBH) * (P // PB // 2)
                                        + b * NBH + nb % NBH, 0)),
            pl.BlockSpec((8, cout), lambda b, nb: (0, 0)),
            pl.BlockSpec((cout, cout), lambda b, nb: (0, 0)),
        ],
        out_specs=pl.BlockSpec((1, cout, PB), lambda b, nb: (b, 0, nb)),
        out_shape=jax.ShapeDtypeStruct((B, cout, N), jnp.float32),
    )(y2, ss, eye)


def _bn_scale_shift(st, gamma, beta):
    mean = st[0] / P
    var = st[1] / P - mean * mean
    s = gamma / jnp.sqrt(var + 1e-5)
    t = beta - mean * s
    return jnp.concatenate(
        [s[None, :], t[None, :], jnp.zeros((6, s.shape[0]), jnp.float32)], axis=0)


def kernel(y_points, y_feats, x_points, x_feats,
           W0, b0, gamma0, beta0,
           W1, b1, gamma1, beta1,
           W2, b2, gamma2, beta2):
    yf_flat = y_feats.reshape(B * M, CY)
    halves = []
    for h in range(2):
        idx_h, w_h = _knn(x_points, y_points, h)
        halves.append(_sc_interp(yf_flat, idx_h, w_h))
    y0, st0 = _layer0(halves[0], halves[1], x_feats.reshape(P, CX),
                      W0[:, :CY].astype(jnp.bfloat16),
                      W0[:, CY:].astype(jnp.bfloat16))
    ss0 = _bn_scale_shift(st0, gamma0, beta0)
    y1, st1 = _layer(y0, ss0, W1.astype(jnp.bfloat16))
    ss1 = _bn_scale_shift(st1, gamma1, beta1)
    y2, st2 = _layer(y1, ss1, W2.astype(jnp.bfloat16))
    ss2 = _bn_scale_shift(st2, gamma2, beta2)
    return _final(y2, ss2, W2.shape[0])


# speedup vs baseline: 14.8369x; 1.0826x over previous
import functools

import jax
import jax.numpy as jnp
from jax import lax
from jax.experimental import pallas as pl
from jax.experimental.pallas import tpu as pltpu
from jax.experimental.pallas import tpu_sc as plsc

B, N, M = 8, 4096, 1024
CY, CX = 256, 128
P = B * N
PB = 512
NB = N // PB
BIG = 3.0e38

NC, NS = 2, 16
NW = NC * NS
QPW = P // NW
T = 32
STEPS = QPW // T


def _split3(v):
    h = v.astype(jnp.bfloat16).astype(jnp.float32)
    r = v - h
    m = r.astype(jnp.bfloat16).astype(jnp.float32)
    return h, m, r - m


def _knn_body(xp_ref, yp_ref, idx_ref, w_ref):
    b = pl.program_id(0)
    xp = xp_ref[0]
    yp = yp_ref[0]
    x2 = jnp.sum(xp * xp, axis=1, keepdims=True)
    y2 = jnp.sum(yp * yp, axis=1, keepdims=True)
    one_x = jnp.ones((PB, 1), jnp.float32)
    one_y = jnp.ones((M, 1), jnp.float32)
    xh, xm, xl = _split3(xp)
    yh, ym, yl = _split3(-2.0 * yp)
    x2h, x2m, x2l = _split3(x2)
    y2h, y2m, y2l = _split3(y2)
    x_cols = [xh, xm, xh, xl, xh, xm, one_x, one_x, one_x, x2h, x2m, x2l]
    y_cols = [yh, yh, ym, yh, yl, ym, y2h, y2m, y2l, one_y, one_y, one_y]
    x_aug = jnp.concatenate(x_cols, axis=1).astype(jnp.bfloat16)
    y_aug = jnp.concatenate(y_cols, axis=1).astype(jnp.bfloat16)
    d2t = lax.dot_general(y_aug, x_aug, (((1,), (1,)), ((), ())),
                          preferred_element_type=jnp.float32)
    d2t = jnp.maximum(d2t, 0.0)
    vals, idxs = [], []
    for k in range(3):
        mk = jnp.min(d2t, axis=0, keepdims=True)
        ik = jnp.argmin(d2t, axis=0).astype(jnp.int32)[None, :]
        vals.append(mk)
        idxs.append(ik)
        if k < 2:
            d2t = jnp.where(d2t <= mk, BIG, d2t)
    r = [1.0 / (v + 1e-8) for v in vals]
    rs = r[0] + r[1] + r[2]
    base = b * M
    one16 = jnp.ones((1, 16), jnp.float32)
    for k in range(3):
        idx_ref[k:k + 1, :] = idxs[k] + base
        w_ref[k] = lax.dot_general(r[k] / rs, one16, (((0,), (0,)), ((), ())),
                                   preferred_element_type=jnp.float32)


NBH = NB // 2
PH = P // 2


def _knn(x_points, y_points, h):
    return pl.pallas_call(
        _knn_body,
        grid=(B, NBH),
        in_specs=[
            pl.BlockSpec((1, PB, 3), lambda b, nb: (b, h * NBH + nb, 0)),
            pl.BlockSpec((1, M, 3), lambda b, nb: (b, 0, 0)),
        ],
        out_specs=[
            pl.BlockSpec((3, PB), lambda b, nb: (0, b * NBH + nb)),
            pl.BlockSpec((3, PB, 16), lambda b, nb: (0, b * NBH + nb, 0)),
        ],
        out_shape=[
            jax.ShapeDtypeStruct((3, PH), jnp.int32),
            jax.ShapeDtypeStruct((3, PH, 16), jnp.float32),
        ],
    )(x_points, y_points)


QPWH = PH // NW
STEPSH = QPWH // T


def _sc_interp_body(yf_ref, idx_ref, w_ref, out_ref, *scr):
    bufs = (scr[0:10], scr[10:20])
    sems = scr[20:22]
    wid = lax.axis_index("s") * NC + lax.axis_index("c")
    base0 = wid * QPWH
    obase0 = base0

    def issue(b, s):
        i0, i1, i2, w0, w1, w2, r0, r1, r2, acc = bufs[b]
        base = base0 + s * T
        pltpu.sync_copy(w_ref.at[0, pl.ds(base, T)], w0)
        pltpu.sync_copy(w_ref.at[1, pl.ds(base, T)], w1)
        pltpu.sync_copy(w_ref.at[2, pl.ds(base, T)], w2)
        pltpu.sync_copy(idx_ref.at[0, pl.ds(base, T)], i0)
        pltpu.sync_copy(idx_ref.at[1, pl.ds(base, T)], i1)
        pltpu.sync_copy(idx_ref.at[2, pl.ds(base, T)], i2)
        pltpu.async_copy(yf_ref.at[i0], r0, sems[b])
        pltpu.async_copy(yf_ref.at[i1], r1, sems[b])
        pltpu.async_copy(yf_ref.at[i2], r2, sems[b])

    def drain(b):
        i0, i1, i2, w0, w1, w2, r0, r1, r2, acc = bufs[b]
        pltpu.make_async_copy(yf_ref.at[i0], r0, sems[b]).wait()
        pltpu.make_async_copy(yf_ref.at[i1], r1, sems[b]).wait()
        pltpu.make_async_copy(yf_ref.at[i2], r2, sems[b]).wait()

    def compute(b, s):
        i0, i1, i2, w0, w1, w2, r0, r1, r2, acc = bufs[b]
        obase = obase0 + s * T

        def q_body(q, carry2):
            wv0 = w0[q, :]
            wv1 = w1[q, :]
            wv2 = w2[q, :]
            for c in range(CY // 16):
                sl = pl.ds(c * 16, 16)
                acc[q, sl] = (r0[q, sl] * wv0 + r1[q, sl] * wv1
                              + r2[q, sl] * wv2)
            return carry2

        lax.fori_loop(0, T, q_body, 0)
        pltpu.sync_copy(acc, out_ref.at[pl.ds(obase, T)])

    issue(0, 0)
    issue(1, 1)

    def step2(t2, carry):
        s = t2 * 2
        for b in range(2):
            drain(b)
            compute(b, s + b)

            @pl.when(s + b + 2 < STEPSH)
            def _():
                issue(b, s + b + 2)
        return carry

    lax.fori_loop(0, STEPSH // 2, step2, 0)


def _sc_interp(yf_flat, idx, w):
    bufset = [
        pltpu.VMEM((T,), jnp.int32),
        pltpu.VMEM((T,), jnp.int32),
        pltpu.VMEM((T,), jnp.int32),
        pltpu.VMEM((T, 16), jnp.float32),
        pltpu.VMEM((T, 16), jnp.float32),
        pltpu.VMEM((T, 16), jnp.float32),
        pltpu.VMEM((T, CY), jnp.float32),
        pltpu.VMEM((T, CY), jnp.float32),
        pltpu.VMEM((T, CY), jnp.float32),
        pltpu.VMEM((T, CY), jnp.float32),
    ]
    kfn = functools.partial(
        pl.kernel,
        out_type=jax.ShapeDtypeStruct((PH, CY), jnp.float32),
        mesh=plsc.VectorSubcoreMesh(core_axis_name="c", subcore_axis_name="s"),
        scratch_types=bufset + bufset
        + [pltpu.SemaphoreType.DMA, pltpu.SemaphoreType.DMA],
    )(_sc_interp_body)
    return kfn(yf_flat, idx, w)


def _l0_body(a0_ref, a1_ref, xf_ref, wa_ref, wb_ref, y_ref, st_ref, acc):
    step = pl.program_id(0)

    @pl.when(step == 0)
    def _():
        acc[...] = jnp.zeros_like(acc)

    def do(a_ref):
        a = a_ref[...].astype(jnp.bfloat16)
        xf = xf_ref[...].astype(jnp.bfloat16)
        y = lax.dot_general(a, wa_ref[...], (((1,), (1,)), ((), ())),
                            preferred_element_type=jnp.float32)
        y += lax.dot_general(xf, wb_ref[...], (((1,), (1,)), ((), ())),
                             preferred_element_type=jnp.float32)
        y_ref[...] = y.astype(jnp.bfloat16)
        acc[0:1, :] += jnp.sum(y, axis=0, keepdims=True)
        acc[1:2, :] += jnp.sum(y * y, axis=0, keepdims=True)

    @pl.when(step < P // PB // 2)
    def _():
        do(a0_ref)

    @pl.when(step >= P // PB // 2)
    def _():
        do(a1_ref)

    @pl.when(step == pl.num_programs(0) - 1)
    def _():
        st_ref[...] = acc[...]


def _layer0(interp0, interp1, xf, W0a, W0b):
    cout = W0a.shape[0]
    nh = P // PB // 2
    return pl.pallas_call(
        _l0_body,
        grid=(P // PB,),
        in_specs=[
            pl.BlockSpec((PB, CY), lambda s: (jnp.minimum(s, nh - 1), 0)),
            pl.BlockSpec((PB, CY), lambda s: (jnp.maximum(s - nh, 0), 0)),
            pl.BlockSpec((PB, CX),
                         lambda s: (((s % nh) // NBH) * NB
                                    + (s // nh) * NBH + s % NBH, 0)),
            pl.BlockSpec((cout, CY), lambda s: (0, 0)),
            pl.BlockSpec((cout, CX), lambda s: (0, 0)),
        ],
        out_specs=[
            pl.BlockSpec((PB, cout), lambda s: (s, 0)),
            pl.BlockSpec((8, cout), lambda s: (0, 0)),
        ],
        out_shape=[
            jax.ShapeDtypeStruct((P, cout), jnp.bfloat16),
            jax.ShapeDtypeStruct((8, cout), jnp.float32),
        ],
        scratch_shapes=[pltpu.VMEM((8, cout), jnp.float32)],
    )(interp0, interp1, xf, W0a, W0b)


def _bn_st(st_ref, g_ref, bt_ref):
    mean = st_ref[0:1, :] * (1.0 / P)
    var = st_ref[1:2, :] * (1.0 / P) - mean * mean
    s = g_ref[...] * lax.rsqrt(var + 1e-5)
    return s, bt_ref[...] - mean * s


def _layer_body(y_ref, st_ref, g_ref, bt_ref, w_ref, out_ref, sto_ref, acc):
    step = pl.program_id(0)

    @pl.when(step == 0)
    def _():
        acc[...] = jnp.zeros_like(acc)

    sc, sh = _bn_st(st_ref, g_ref, bt_ref)
    yin = y_ref[...].astype(jnp.float32)
    a = jnp.maximum(yin * sc + sh, 0.0)
    y = lax.dot_general(a.astype(jnp.bfloat16), w_ref[...],
                        (((1,), (1,)), ((), ())),
                        preferred_element_type=jnp.float32)
    out_ref[...] = y.astype(jnp.bfloat16)
    acc[0:1, :] += jnp.sum(y, axis=0, keepdims=True)
    acc[1:2, :] += jnp.sum(y * y, axis=0, keepdims=True)

    @pl.when(step == pl.num_programs(0) - 1)
    def _():
        sto_ref[...] = acc[...]


def _layer(y, st, gamma, beta, W):
    cout, cin = W.shape
    return pl.pallas_call(
        _layer_body,
        grid=(P // PB,),
        in_specs=[
            pl.BlockSpec((PB, cin), lambda s: (s, 0)),
            pl.BlockSpec((8, cin), lambda s: (0, 0)),
            pl.BlockSpec((1, cin), lambda s: (0, 0)),
            pl.BlockSpec((1, cin), lambda s: (0, 0)),
            pl.BlockSpec((cout, cin), lambda s: (0, 0)),
        ],
        out_specs=[
            pl.BlockSpec((PB, cout), lambda s: (s, 0)),
            pl.BlockSpec((8, cout), lambda s: (0, 0)),
        ],
        out_shape=[
            jax.ShapeDtypeStruct((P, cout), jnp.bfloat16),
            jax.ShapeDtypeStruct((8, cout), jnp.float32),
        ],
        scratch_shapes=[pltpu.VMEM((8, cout), jnp.float32)],
    )(y, st, gamma.reshape(1, cin), beta.reshape(1, cin), W)


def _final_body(y_ref, st_ref, g_ref, bt_ref, eye_ref, out_ref):
    sc, sh = _bn_st(st_ref, g_ref, bt_ref)
    yin = y_ref[...].astype(jnp.float32)
    a = jnp.maximum(yin * sc + sh, 0.0)
    out_ref[0] = lax.dot_general(eye_ref[...], a, (((1,), (1,)), ((), ())),
                                 preferred_element_type=jnp.float32)


def _final(y2, st, gamma, beta, cout):
    eye = jnp.eye(cout, dtype=jnp.float32)
    return pl.pallas_call(
        _final_body,
        grid=(B, NB),
        in_specs=[
            pl.BlockSpec((PB, cout),
                         lambda b, nb: ((nb // NBH) * (P // PB // 2)
                                        + b * NBH + nb % NBH, 0)),
            pl.BlockSpec((8, cout), lambda b, nb: (0, 0)),
            pl.BlockSpec((1, cout), lambda b, nb: (0, 0)),
            pl.BlockSpec((1, cout), lambda b, nb: (0, 0)),
            pl.BlockSpec((cout, cout), lambda b, nb: (0, 0)),
        ],
        out_specs=pl.BlockSpec((1, cout, PB), lambda b, nb: (b, 0, nb)),
        out_shape=jax.ShapeDtypeStruct((B, cout, N), jnp.float32),
    )(y2, st, gamma.reshape(1, cout), beta.reshape(1, cout), eye)


def kernel(y_points, y_feats, x_points, x_feats,
           W0, b0, gamma0, beta0,
           W1, b1, gamma1, beta1,
           W2, b2, gamma2, beta2):
    yf_flat = y_feats.reshape(B * M, CY)
    halves = []
    for h in range(2):
        idx_h, w_h = _knn(x_points, y_points, h)
        halves.append(_sc_interp(yf_flat, idx_h, w_h))
    y0, st0 = _layer0(halves[0], halves[1], x_feats.reshape(P, CX),
                      W0[:, :CY].astype(jnp.bfloat16),
                      W0[:, CY:].astype(jnp.bfloat16))
    y1, st1 = _layer(y0, st0, gamma0, beta0, W1.astype(jnp.bfloat16))
    y2, st2 = _layer(y1, st1, gamma1, beta1, W2.astype(jnp.bfloat16))
    return _final(y2, st2, gamma2, beta2, W2.shape[0])
